# baseline (speedup 1.0000x reference)
import functools

import jax
import jax.numpy as jnp
from jax import lax
from jax.experimental import pallas as pl
from jax.experimental.pallas import tpu as pltpu
from jax.experimental.pallas import tpu_sc as plsc

_R = 4


def _linear_kernel(x_ref, w_ref, b_ref, y_ref):
    y_ref[...] = jax.lax.dot_general(
        x_ref[...], w_ref[...], (((1,), (1,)), ((), ())),
        preferred_element_type=jnp.float32) + b_ref[...]


def _linear(x, W, b):
    n, c_in = x.shape
    c_out = W.shape[0]
    bn = 10000
    return pl.pallas_call(
        _linear_kernel,
        grid=(n // bn,),
        in_specs=[
            pl.BlockSpec((bn, c_in), lambda i: (i, i - i)),
            pl.BlockSpec((c_out, c_in), lambda i: (i - i, i - i)),
            pl.BlockSpec((1, c_out), lambda i: (i - i, i - i)),
        ],
        out_specs=pl.BlockSpec((bn, c_out), lambda i: (i, i - i)),
        out_shape=jax.ShapeDtypeStruct((n, c_out), jnp.float32),
    )(x, W, b.reshape(1, c_out))


def _make_dup_sc(n_out, c):
    info = plsc.get_sparse_core_info()
    nw = info.num_cores * info.num_subcores
    ch = 64
    n_chunks = n_out // ch
    per_w = -(-n_chunks // nw)
    mesh = plsc.VectorSubcoreMesh(core_axis_name="c", subcore_axis_name="s")

    @functools.partial(
        pl.kernel, mesh=mesh,
        out_type=jax.ShapeDtypeStruct((n_out, c), jnp.float32),
        scratch_types=[
            pltpu.VMEM((ch,), jnp.int32),
            pltpu.VMEM((ch, c), jnp.float32),
            pltpu.SemaphoreType.DMA,
        ],
    )
    def dup(y_hbm, idx_hbm, out_hbm, idx_v, rows_v, sem):
        wid = lax.axis_index("s") * info.num_cores + lax.axis_index("c")

        def body(t, carry):
            k = t * nw + wid

            @pl.when(k < n_chunks)
            def _():
                base = k * ch
                pltpu.sync_copy(idx_hbm.at[pl.ds(base, ch)], idx_v)
                pltpu.async_copy(y_hbm.at[idx_v], rows_v, sem).wait()
                pltpu.sync_copy(rows_v, out_hbm.at[pl.ds(base, ch)])
            return carry

        lax.fori_loop(jnp.int32(0), jnp.int32(per_w), body, jnp.int32(0))

    return dup


def kernel(x, edge_index, W, b):
    n, c_in = x.shape
    c_out = W.shape[0]
    y = _linear(x, W, b)
    idx = jnp.arange(_R * n, dtype=jnp.int32) // _R
    dup = _make_dup_sc(_R * n, c_out)
    return dup(y, idx)


# speedup vs baseline: 6.5043x; 6.5043x over previous
import jax
import jax.numpy as jnp
from jax.experimental import pallas as pl
from jax.experimental.pallas import tpu as pltpu

_R = 4


def _dup_linear_kernel(x_ref, w_ref, b_ref, o_ref):
    y = jax.lax.dot_general(
        x_ref[...], w_ref[...], (((1,), (1,)), ((), ())),
        preferred_element_type=jnp.float32)
    y = y + b_ref[...]
    for r in range(_R):
        o_ref[r::_R, :] = y


def kernel(x, edge_index, W, b):
    n, c_in = x.shape
    c_out = W.shape[0]
    b2 = b.reshape(1, c_out)

    bn = 10000
    grid = (n // bn,)
    out = pl.pallas_call(
        _dup_linear_kernel,
        grid=grid,
        in_specs=[
            pl.BlockSpec((bn, c_in), lambda i: (i, i - i)),
            pl.BlockSpec((c_out, c_in), lambda i: (i - i, i - i)),
            pl.BlockSpec((1, c_out), lambda i: (i - i, i - i)),
        ],
        out_specs=pl.BlockSpec((_R * bn, c_out), lambda i: (i, i - i)),
        out_shape=jax.ShapeDtypeStruct((_R * n, c_out), jnp.float32),
        compiler_params=pltpu.CompilerParams(
            dimension_semantics=("parallel",)),
    )(x, W, b2)
    return out
